# Initial kernel scaffold; baseline (speedup 1.0000x reference)
import jax
import jax.numpy as jnp
from jax.experimental import pallas as pl


def kernel(value, depth, position, table, W, b):
    raise NotImplementedError("write your pallas kernel here")


# speedup vs baseline: 1.7631x; 1.7631x over previous
import functools

import jax
import jax.numpy as jnp
from jax import lax
from jax.experimental import pallas as pl
from jax.experimental.pallas import tpu as pltpu
from jax.experimental.pallas import tpu_sc as plsc

NUM_VOCAB = 100000
EMBED_DIM = 64
N, S, A = 4096, 50, 3
B = N * S

NC = 2
NS = 16
NW = NC * NS
B_PER_W = B // NW
SUB = 128
CHUNK = 640
N_SUB = CHUNK // SUB
N_CHUNK = B_PER_W // CHUNK
ROWS_PER_W = B_PER_W // SUB


def _sc_gather(value_flat, table):
    mesh = plsc.VectorSubcoreMesh(core_axis_name="c", subcore_axis_name="s")

    @functools.partial(
        pl.kernel,
        mesh=mesh,
        out_type=jax.ShapeDtypeStruct((B, EMBED_DIM), jnp.float32),
        compiler_params=pltpu.CompilerParams(use_tc_tiling_on_sc=False),
        scratch_types=[
            pltpu.VMEM((B_PER_W,), jnp.int32),
            pltpu.VMEM((CHUNK, EMBED_DIM), jnp.float32),
            pltpu.SemaphoreType.DMA,
        ],
    )
    def gather_kernel(value_hbm, table_hbm, out_hbm, idx_v, rows_v, sem):
        wid = lax.axis_index("s") * NC + lax.axis_index("c")
        base = wid * B_PER_W
        pltpu.sync_copy(value_hbm.at[pl.ds(base, B_PER_W)], idx_v)

        def chunk_body(j, carry):
            copies = []
            for k in range(N_SUB):
                copies.append(
                    pltpu.async_copy(
                        table_hbm.at[idx_v.at[pl.ds(j * CHUNK + k * SUB, SUB)]],
                        rows_v.at[pl.ds(k * SUB, SUB)],
                        sem,
                    )
                )
            for c in copies:
                c.wait()
            pltpu.sync_copy(
                rows_v, out_hbm.at[pl.ds(base + j * CHUNK, CHUNK)]
            )
            return carry

        lax.fori_loop(0, N_CHUNK, chunk_body, 0)

    return gather_kernel(value_flat, table)


def _tc_add(gathered, position2d, W, b2d):
    BN = 2048
    grid = (B // BN,)

    def add_kernel(g_ref, p_ref, w_ref, b_ref, o_ref):
        lin = jnp.dot(p_ref[...], w_ref[...], preferred_element_type=jnp.float32)
        o_ref[...] = g_ref[...] + lin + b_ref[...]

    return pl.pallas_call(
        add_kernel,
        grid=grid,
        in_specs=[
            pl.BlockSpec((BN, EMBED_DIM), lambda i: (i, 0)),
            pl.BlockSpec((BN, A), lambda i: (i, 0)),
            pl.BlockSpec((A, EMBED_DIM), lambda i: (0, 0)),
            pl.BlockSpec((1, EMBED_DIM), lambda i: (0, 0)),
        ],
        out_specs=pl.BlockSpec((BN, EMBED_DIM), lambda i: (i, 0)),
        out_shape=jax.ShapeDtypeStruct((B, EMBED_DIM), jnp.float32),
    )(gathered, position2d, W, b2d)


def kernel(value, depth, position, table, W, b):
    del depth
    gathered = _sc_gather(value.reshape(B), table)
    out2d = _tc_add(gathered, position.reshape(B, A), W, b.reshape(1, EMBED_DIM))
    return out2d.reshape(N, S, EMBED_DIM)


# speedup vs baseline: 2.8158x; 1.5971x over previous
import functools

import jax
import jax.numpy as jnp
from jax import lax
from jax.experimental import pallas as pl
from jax.experimental.pallas import tpu as pltpu
from jax.experimental.pallas import tpu_sc as plsc

NUM_VOCAB = 100000
EMBED_DIM = 64
N, S, A = 4096, 50, 3
B = N * S
L = 16
EC = EMBED_DIM // L

NC = 2
NS = 16
NW = NC * NS
B_PER_W = B // NW
SUB = 128
CHUNK = 640
N_SUB = CHUNK // SUB
N_CHUNK = B_PER_W // CHUNK


def _sc_fused(value_flat, table, pos_t, wb):
    mesh = plsc.VectorSubcoreMesh(core_axis_name="c", subcore_axis_name="s")

    @functools.partial(
        pl.kernel,
        mesh=mesh,
        out_type=jax.ShapeDtypeStruct((B, EMBED_DIM), jnp.float32),
        compiler_params=pltpu.CompilerParams(
            use_tc_tiling_on_sc=False, needs_layout_passes=False),
        scratch_types=[
            pltpu.VMEM((B_PER_W,), jnp.int32),
            pltpu.VMEM((B_PER_W,), jnp.float32),
            pltpu.VMEM((B_PER_W,), jnp.float32),
            pltpu.VMEM((B_PER_W,), jnp.float32),
            pltpu.VMEM((A + 1, EMBED_DIM), jnp.float32),
            pltpu.VMEM((CHUNK, EMBED_DIM), jnp.float32),
            pltpu.VMEM((CHUNK, EMBED_DIM), jnp.float32),
            pltpu.SemaphoreType.DMA,
            pltpu.SemaphoreType.DMA,
            pltpu.SemaphoreType.DMA,
            pltpu.SemaphoreType.DMA,
        ],
    )
    def fused_kernel(value_hbm, table_hbm, pos_hbm, wb_hbm, out_hbm,
                     idx_v, p0_v, p1_v, p2_v, wb_v, rows_a, rows_b,
                     gsem_a, gsem_b, osem_a, osem_b):
        wid = lax.axis_index("s") * NC + lax.axis_index("c")
        base = wid * B_PER_W
        p_v = (p0_v, p1_v, p2_v)
        pltpu.sync_copy(value_hbm.at[pl.ds(base, B_PER_W)], idx_v)
        for a in range(A):
            pltpu.sync_copy(pos_hbm.at[pl.ds(a * B + base, B_PER_W)], p_v[a])
        pltpu.sync_copy(wb_hbm, wb_v)
        rows = (rows_a, rows_b)
        gsem = (gsem_a, gsem_b)
        osem = (osem_a, osem_b)

        def fire_gathers(j, buf, sem):
            return [
                pltpu.async_copy(
                    table_hbm.at[idx_v.at[pl.ds(j * CHUNK + k * SUB, SUB)]],
                    buf.at[pl.ds(k * SUB, SUB)],
                    sem,
                )
                for k in range(N_SUB)
            ]

        def add_linear(j, buf):
            w = [[wb_v[a, pl.ds(k * L, L)] for k in range(EC)] for a in range(A)]
            bb = [wb_v[A, pl.ds(k * L, L)] for k in range(EC)]

            def body(i, carry):
                src = jnp.full((L,), j * CHUNK + i, jnp.int32)
                p0 = plsc.load_gather(p_v[0], [src])
                p1 = plsc.load_gather(p_v[1], [src])
                p2 = plsc.load_gather(p_v[2], [src])
                for k in range(EC):
                    lin = bb[k] + p0 * w[0][k] + p1 * w[1][k] + p2 * w[2][k]
                    buf[i, pl.ds(k * L, L)] += lin
                return carry

            lax.fori_loop(0, CHUNK, body, 0)

        pending_g = fire_gathers(0, rows[0], gsem[0])
        pending_o = [None, None]
        for j in range(N_CHUNK):
            cur = j % 2
            nxt = (j + 1) % 2
            if j + 1 < N_CHUNK:
                if pending_o[nxt] is not None:
                    pending_o[nxt].wait()
                    pending_o[nxt] = None
                next_g = fire_gathers(j + 1, rows[nxt], gsem[nxt])
            for c in pending_g:
                c.wait()
            add_linear(j, rows[cur])
            pending_o[cur] = pltpu.async_copy(
                rows[cur], out_hbm.at[pl.ds(base + j * CHUNK, CHUNK)], osem[cur]
            )
            if j + 1 < N_CHUNK:
                pending_g = next_g
        for o in pending_o:
            if o is not None:
                o.wait()

    return fused_kernel(value_flat, table, pos_t, wb)


def kernel(value, depth, position, table, W, b):
    del depth
    pos_t = position.reshape(B, A).T.reshape(A * B)
    wb = jnp.concatenate([W, b.reshape(1, EMBED_DIM)], axis=0)
    out2d = _sc_fused(value.reshape(B), table, pos_t, wb)
    return out2d.reshape(N, S, EMBED_DIM)


# speedup vs baseline: 3.3609x; 1.1936x over previous
import functools

import jax
import jax.numpy as jnp
from jax import lax
from jax.experimental import pallas as pl
from jax.experimental.pallas import tpu as pltpu
from jax.experimental.pallas import tpu_sc as plsc

NUM_VOCAB = 100000
EMBED_DIM = 64
N, S, A = 4096, 50, 3
B = N * S
L = 16

NC = 2
NS = 16
NW = NC * NS
B_PER_W = B // NW
SUB = 128
CHUNK = 640
N_SUB = CHUNK // SUB
N_CHUNK = B_PER_W // CHUNK


def _sc_gather(value_flat, table):
    mesh = plsc.VectorSubcoreMesh(core_axis_name="c", subcore_axis_name="s")

    @functools.partial(
        pl.kernel,
        mesh=mesh,
        out_type=jax.ShapeDtypeStruct((B, EMBED_DIM), jnp.float32),
        compiler_params=pltpu.CompilerParams(use_tc_tiling_on_sc=False),
        scratch_types=[
            pltpu.VMEM((B_PER_W,), jnp.int32),
            pltpu.VMEM((CHUNK, EMBED_DIM), jnp.float32),
            pltpu.VMEM((CHUNK, EMBED_DIM), jnp.float32),
            pltpu.SemaphoreType.DMA,
            pltpu.SemaphoreType.DMA,
            pltpu.SemaphoreType.DMA,
            pltpu.SemaphoreType.DMA,
        ],
    )
    def gather_kernel(value_hbm, table_hbm, out_hbm,
                      idx_v, rows_a, rows_b, gsem_a, gsem_b, osem_a, osem_b):
        wid = lax.axis_index("s") * NC + lax.axis_index("c")
        base = wid * B_PER_W
        pltpu.sync_copy(value_hbm.at[pl.ds(base, B_PER_W)], idx_v)
        rows = (rows_a, rows_b)
        gsem = (gsem_a, gsem_b)
        osem = (osem_a, osem_b)

        def fire_gathers(j, buf, sem):
            return [
                pltpu.async_copy(
                    table_hbm.at[idx_v.at[pl.ds(j * CHUNK + k * SUB, SUB)]],
                    buf.at[pl.ds(k * SUB, SUB)],
                    sem,
                )
                for k in range(N_SUB)
            ]

        pending_g = fire_gathers(0, rows[0], gsem[0])
        pending_o = [None, None]
        for j in range(N_CHUNK):
            cur = j % 2
            nxt = (j + 1) % 2
            if j + 1 < N_CHUNK:
                if pending_o[nxt] is not None:
                    pending_o[nxt].wait()
                    pending_o[nxt] = None
                next_g = fire_gathers(j + 1, rows[nxt], gsem[nxt])
            for c in pending_g:
                c.wait()
            pending_o[cur] = pltpu.async_copy(
                rows[cur], out_hbm.at[pl.ds(base + j * CHUNK, CHUNK)], osem[cur]
            )
            if j + 1 < N_CHUNK:
                pending_g = next_g
        for o in pending_o:
            if o is not None:
                o.wait()

    return gather_kernel(value_flat, table)


H = N // 2


def _tc_add_transpose(g2, p2, W2):
    RPS = N * EMBED_DIM // 128

    def add_t_kernel(g_ref, p_ref, w_ref, o_ref):
        x = g_ref[...]
        lin = jnp.dot(p_ref[0], w_ref[...],
                      preferred_element_type=jnp.float32,
                      precision=jax.lax.Precision.HIGHEST)
        y = (x + lin).T
        o_ref[0, :, 0:H] = y[0:EMBED_DIM, :]
        o_ref[0, :, H:N] = y[EMBED_DIM:128, :]

    return pl.pallas_call(
        add_t_kernel,
        grid=(S,),
        in_specs=[
            pl.BlockSpec((RPS, 128), lambda i: (i, 0)),
            pl.BlockSpec((1, H, 8), lambda i: (i, 0, 0)),
            pl.BlockSpec((8, 128), lambda i: (0, 0)),
        ],
        out_specs=pl.BlockSpec((1, EMBED_DIM, N), lambda i: (i, 0, 0)),
        out_shape=jax.ShapeDtypeStruct((S, EMBED_DIM, N), jnp.float32),
    )(g2, p2, W2)


def kernel(value, depth, position, table, W, b):
    del depth
    value_t = value.T
    value_flat = jnp.stack(
        [value_t[:, :H], value_t[:, H:]], axis=2).reshape(B)
    gathered = _sc_gather(value_flat, table)
    g2 = gathered.reshape(B * EMBED_DIM // 128, 128)
    pos_s = position.transpose(1, 0, 2)
    ones = jnp.ones((S, H, 1), jnp.float32)
    p2 = jnp.concatenate(
        [pos_s[:, :H, :], ones, pos_s[:, H:, :], ones], axis=2)
    z = jnp.zeros_like(W)
    zb = jnp.zeros_like(b)
    W2 = jnp.concatenate([
        jnp.concatenate([W, z], axis=1),
        jnp.concatenate([b.reshape(1, -1), zb.reshape(1, -1)], axis=1),
        jnp.concatenate([z, W], axis=1),
        jnp.concatenate([zb.reshape(1, -1), b.reshape(1, -1)], axis=1),
    ], axis=0)
    out_t = _tc_add_transpose(g2, p2, W2)
    return jnp.transpose(out_t, (2, 0, 1))
